# baseline (speedup 1.0000x reference)
import functools

import jax
import jax.numpy as jnp
from jax import lax
from jax.experimental import pallas as pl
from jax.experimental.pallas import tpu as pltpu

KS = 3
PAD = 128
NCHUNK = 2


def _resblock_kernel(x_ref, w1_ref, b1_ref, w2_ref, b2_ref,
                     o_ref, s1_ref, s2_ref, sx_ref, *, H, W):
    C = x_ref.shape[2]
    W2 = 2 * W
    L = H * W2
    CH = L // NCHUNK
    HH = CH // W2
    bf16 = jnp.bfloat16

    lmod = lax.broadcasted_iota(jnp.int32, (1, CH), 1) % W
    ok_left = lmod != 0
    ok_right = lmod != (W - 1)

    zb = jnp.zeros((3 * C, PAD), bf16)
    for s in (s1_ref, s2_ref):
        s[:, :PAD] = zb
        s[:, PAD + L:] = zb

    def build_shifted(s_ref):
        for j in range(NCHUNK):
            lo = PAD + j * CH
            left = jnp.where(ok_left, s_ref[C:2 * C, lo - 1:lo - 1 + CH],
                             bf16(0))
            right = jnp.where(ok_right, s_ref[C:2 * C, lo + 1:lo + 1 + CH],
                              bf16(0))
            s_ref[:C, lo:lo + CH] = left
            s_ref[2 * C:, lo:lo + CH] = right

    def conv_chunk(w_ref, b_ref, s_ref, j):
        lo = PAD + j * CH
        acc = jnp.zeros((C, CH), jnp.float32)
        for dy in range(KS):
            off = (dy - 1) * W2
            acc = acc + jnp.dot(w_ref[dy], s_ref[:, lo + off:lo + off + CH],
                                preferred_element_type=jnp.float32)
        return acc + b_ref[...]

    for j in range(NCHUNK):
        xa = x_ref[0, 0, :, j * HH:(j + 1) * HH, :]
        xb = x_ref[0, 1, :, j * HH:(j + 1) * HH, :]
        xpk = jnp.concatenate([xa, xb], axis=2).reshape(C, CH)
        sx_ref[:, j * CH:(j + 1) * CH] = xpk
        s1_ref[C:2 * C, PAD + j * CH:PAD + (j + 1) * CH] = xpk.astype(bf16)

    build_shifted(s1_ref)
    for j in range(NCHUNK):
        h = jnp.maximum(conv_chunk(w1_ref, b1_ref, s1_ref, j), 0.0)
        s2_ref[C:2 * C, PAD + j * CH:PAD + (j + 1) * CH] = h.astype(bf16)

    build_shifted(s2_ref)
    for j in range(NCHUNK):
        y = conv_chunk(w2_ref, b2_ref, s2_ref, j)
        y = jnp.maximum(y + sx_ref[:, j * CH:(j + 1) * CH], 0.0)
        y3 = y.reshape(C, HH, W2)
        o_ref[0, 0, :, j * HH:(j + 1) * HH, :] = y3[:, :, :W]
        o_ref[0, 1, :, j * HH:(j + 1) * HH, :] = y3[:, :, W:]


def kernel(x_nchw, w1, b1, w2, b2):
    N, C, H, W = x_nchw.shape
    assert w2.shape[-1] == C and N % 2 == 0 and 2 * W % 128 == 0
    assert PAD >= 2 * W and (H * 2 * W) % (NCHUNK * 2 * W) == 0

    NB = N // 2
    x_pairs = x_nchw.reshape(NB, 2, C, H, W)

    def stack_w(w):
        t = jnp.transpose(w, (0, 1, 3, 2))
        t = jnp.transpose(t, (0, 2, 1, 3))
        return t.reshape(KS, C, KS * C).astype(jnp.bfloat16)

    w1_s = stack_w(w1)
    w2_s = stack_w(w2)
    b1_c = b1.reshape(C, 1).astype(jnp.float32)
    b2_c = b2.reshape(C, 1).astype(jnp.float32)

    body = functools.partial(_resblock_kernel, H=H, W=W)
    pair_spec = pl.BlockSpec((1, 2, C, H, W), lambda n: (n, 0, 0, 0, 0))
    span = H * 2 * W + 2 * PAD

    out = pl.pallas_call(
        body,
        out_shape=jax.ShapeDtypeStruct((NB, 2, C, H, W), x_nchw.dtype),
        grid_spec=pltpu.PrefetchScalarGridSpec(
            num_scalar_prefetch=0,
            grid=(NB,),
            in_specs=[
                pair_spec,
                pl.BlockSpec((KS, C, KS * C), lambda n: (0, 0, 0)),
                pl.BlockSpec((C, 1), lambda n: (0, 0)),
                pl.BlockSpec((KS, C, KS * C), lambda n: (0, 0, 0)),
                pl.BlockSpec((C, 1), lambda n: (0, 0)),
            ],
            out_specs=pair_spec,
            scratch_shapes=[
                pltpu.VMEM((3 * C, span), jnp.bfloat16),
                pltpu.VMEM((3 * C, span), jnp.bfloat16),
                pltpu.VMEM((C, H * 2 * W), jnp.float32),
            ],
        ),
        compiler_params=pltpu.CompilerParams(
            dimension_semantics=("parallel",)),
    )(x_pairs, w1_s, b1_c, w2_s, b2_c)

    return out.reshape(N, C, H, W)


# speedup vs baseline: 1.0724x; 1.0724x over previous
import functools

import jax
import jax.numpy as jnp
from jax import lax
from jax.experimental import pallas as pl
from jax.experimental.pallas import tpu as pltpu

KS = 3
PAD = 128
NCHUNK = 8


def _resblock_kernel(x_ref, w1_ref, b1_ref, w2_ref, b2_ref,
                     o_ref, s1_ref, s2_ref, sx_ref, *, H, W):
    C = x_ref.shape[2]
    W2 = 2 * W
    L = H * W2
    CH = L // NCHUNK
    HH = CH // W2
    bf16 = jnp.bfloat16

    lmod = lax.broadcasted_iota(jnp.int32, (1, CH), 1) % W
    ok_left = lmod != 0
    ok_right = lmod != (W - 1)

    zb = jnp.zeros((3 * C, PAD), bf16)
    for s in (s1_ref, s2_ref):
        s[:, :PAD] = zb
        s[:, PAD + L:] = zb

    def build_shifted(s_ref):
        for j in range(NCHUNK):
            lo = PAD + j * CH
            left = jnp.where(ok_left, s_ref[C:2 * C, lo - 1:lo - 1 + CH],
                             bf16(0))
            right = jnp.where(ok_right, s_ref[C:2 * C, lo + 1:lo + 1 + CH],
                              bf16(0))
            s_ref[:C, lo:lo + CH] = left
            s_ref[2 * C:, lo:lo + CH] = right

    def conv_chunk(w_ref, b_ref, s_ref, j):
        lo = PAD + j * CH
        acc = jnp.zeros((C, CH), jnp.float32)
        for dy in range(KS):
            off = (dy - 1) * W2
            acc = acc + jnp.dot(w_ref[dy], s_ref[:, lo + off:lo + off + CH],
                                preferred_element_type=jnp.float32)
        return acc + b_ref[...]

    for j in range(NCHUNK):
        xa = x_ref[0, 0, :, j * HH:(j + 1) * HH, :]
        xb = x_ref[0, 1, :, j * HH:(j + 1) * HH, :]
        xpk = jnp.concatenate([xa, xb], axis=2).reshape(C, CH)
        sx_ref[:, j * CH:(j + 1) * CH] = xpk
        s1_ref[C:2 * C, PAD + j * CH:PAD + (j + 1) * CH] = xpk.astype(bf16)

    build_shifted(s1_ref)
    for j in range(NCHUNK):
        h = jnp.maximum(conv_chunk(w1_ref, b1_ref, s1_ref, j), 0.0)
        s2_ref[C:2 * C, PAD + j * CH:PAD + (j + 1) * CH] = h.astype(bf16)

    build_shifted(s2_ref)
    for j in range(NCHUNK):
        y = conv_chunk(w2_ref, b2_ref, s2_ref, j)
        y = jnp.maximum(y + sx_ref[:, j * CH:(j + 1) * CH], 0.0)
        y3 = y.reshape(C, HH, W2)
        o_ref[0, 0, :, j * HH:(j + 1) * HH, :] = y3[:, :, :W]
        o_ref[0, 1, :, j * HH:(j + 1) * HH, :] = y3[:, :, W:]


def kernel(x_nchw, w1, b1, w2, b2):
    N, C, H, W = x_nchw.shape
    assert w2.shape[-1] == C and N % 2 == 0 and 2 * W % 128 == 0
    assert PAD >= 2 * W and (H * 2 * W) % (NCHUNK * 2 * W) == 0

    NB = N // 2
    x_pairs = x_nchw.reshape(NB, 2, C, H, W)

    def stack_w(w):
        t = jnp.transpose(w, (0, 1, 3, 2))
        t = jnp.transpose(t, (0, 2, 1, 3))
        return t.reshape(KS, C, KS * C).astype(jnp.bfloat16)

    w1_s = stack_w(w1)
    w2_s = stack_w(w2)
    b1_c = b1.reshape(C, 1).astype(jnp.float32)
    b2_c = b2.reshape(C, 1).astype(jnp.float32)

    body = functools.partial(_resblock_kernel, H=H, W=W)
    pair_spec = pl.BlockSpec((1, 2, C, H, W), lambda n: (n, 0, 0, 0, 0))
    span = H * 2 * W + 2 * PAD

    out = pl.pallas_call(
        body,
        out_shape=jax.ShapeDtypeStruct((NB, 2, C, H, W), x_nchw.dtype),
        grid_spec=pltpu.PrefetchScalarGridSpec(
            num_scalar_prefetch=0,
            grid=(NB,),
            in_specs=[
                pair_spec,
                pl.BlockSpec((KS, C, KS * C), lambda n: (0, 0, 0)),
                pl.BlockSpec((C, 1), lambda n: (0, 0)),
                pl.BlockSpec((KS, C, KS * C), lambda n: (0, 0, 0)),
                pl.BlockSpec((C, 1), lambda n: (0, 0)),
            ],
            out_specs=pair_spec,
            scratch_shapes=[
                pltpu.VMEM((3 * C, span), jnp.bfloat16),
                pltpu.VMEM((3 * C, span), jnp.bfloat16),
                pltpu.VMEM((C, H * 2 * W), jnp.float32),
            ],
        ),
        compiler_params=pltpu.CompilerParams(
            dimension_semantics=("parallel",)),
    )(x_pairs, w1_s, b1_c, w2_s, b2_c)

    return out.reshape(N, C, H, W)


# speedup vs baseline: 1.1292x; 1.0530x over previous
import functools

import jax
import jax.numpy as jnp
from jax import lax
from jax.experimental import pallas as pl
from jax.experimental.pallas import tpu as pltpu

KS = 3
PAD = 128
NCHUNK = 4


def _resblock_kernel(x_ref, w1_ref, b1_ref, w2_ref, b2_ref,
                     o_ref, s1_ref, s2_ref, sx_ref, *, H, W):
    C = x_ref.shape[2]
    W2 = 2 * W
    L = H * W2
    CH = L // NCHUNK
    HH = CH // W2
    bf16 = jnp.bfloat16

    lmod = lax.broadcasted_iota(jnp.int32, (1, CH), 1) % W
    ok_left = lmod != 0
    ok_right = lmod != (W - 1)

    zb = jnp.zeros((3 * C, PAD), bf16)
    for s in (s1_ref, s2_ref):
        s[:, :PAD] = zb
        s[:, PAD + L:] = zb

    def build_shifted(s_ref):
        for j in range(NCHUNK):
            lo = PAD + j * CH
            left = jnp.where(ok_left, s_ref[C:2 * C, lo - 1:lo - 1 + CH],
                             bf16(0))
            right = jnp.where(ok_right, s_ref[C:2 * C, lo + 1:lo + 1 + CH],
                              bf16(0))
            s_ref[:C, lo:lo + CH] = left
            s_ref[2 * C:, lo:lo + CH] = right

    def conv_chunk(w_ref, b_ref, s_ref, j):
        lo = PAD + j * CH
        acc = jnp.zeros((C, CH), jnp.float32)
        for dy in range(KS):
            off = (dy - 1) * W2
            acc = acc + jnp.dot(w_ref[dy], s_ref[:, lo + off:lo + off + CH],
                                preferred_element_type=jnp.float32)
        return acc + b_ref[...]

    for p in range(x_ref.shape[0]):
        for j in range(NCHUNK):
            xa = x_ref[p, 0, :, j * HH:(j + 1) * HH, :]
            xb = x_ref[p, 1, :, j * HH:(j + 1) * HH, :]
            xpk = jnp.concatenate([xa, xb], axis=2).reshape(C, CH)
            sx_ref[:, j * CH:(j + 1) * CH] = xpk
            s1_ref[C:2 * C, PAD + j * CH:PAD + (j + 1) * CH] = xpk.astype(bf16)

        build_shifted(s1_ref)
        for j in range(NCHUNK):
            h = jnp.maximum(conv_chunk(w1_ref, b1_ref, s1_ref, j), 0.0)
            s2_ref[C:2 * C, PAD + j * CH:PAD + (j + 1) * CH] = h.astype(bf16)

        build_shifted(s2_ref)
        for j in range(NCHUNK):
            y = conv_chunk(w2_ref, b2_ref, s2_ref, j)
            y = jnp.maximum(y + sx_ref[:, j * CH:(j + 1) * CH], 0.0)
            y3 = y.reshape(C, HH, W2)
            o_ref[p, 0, :, j * HH:(j + 1) * HH, :] = y3[:, :, :W]
            o_ref[p, 1, :, j * HH:(j + 1) * HH, :] = y3[:, :, W:]


def kernel(x_nchw, w1, b1, w2, b2):
    N, C, H, W = x_nchw.shape
    assert w2.shape[-1] == C and N % 2 == 0 and 2 * W % 128 == 0
    assert PAD >= 2 * W and (H * 2 * W) % (NCHUNK * 2 * W) == 0

    NB = N // 2
    P = 2 if NB % 2 == 0 else 1
    x_pairs = x_nchw.reshape(NB, 2, C, H, W)

    def stack_w(w):
        t = jnp.transpose(w, (0, 1, 3, 2))
        t = jnp.transpose(t, (0, 2, 1, 3))
        return t.reshape(KS, C, KS * C).astype(jnp.bfloat16)

    w1_s = stack_w(w1)
    w2_s = stack_w(w2)
    b1_c = b1.reshape(C, 1).astype(jnp.float32)
    b2_c = b2.reshape(C, 1).astype(jnp.float32)

    body = functools.partial(_resblock_kernel, H=H, W=W)
    pair_spec = pl.BlockSpec((P, 2, C, H, W), lambda n: (n, 0, 0, 0, 0))
    span = H * 2 * W + 2 * PAD

    out = pl.pallas_call(
        body,
        out_shape=jax.ShapeDtypeStruct((NB, 2, C, H, W), x_nchw.dtype),
        grid_spec=pltpu.PrefetchScalarGridSpec(
            num_scalar_prefetch=0,
            grid=(NB // P,),
            in_specs=[
                pair_spec,
                pl.BlockSpec((KS, C, KS * C), lambda n: (0, 0, 0)),
                pl.BlockSpec((C, 1), lambda n: (0, 0)),
                pl.BlockSpec((KS, C, KS * C), lambda n: (0, 0, 0)),
                pl.BlockSpec((C, 1), lambda n: (0, 0)),
            ],
            out_specs=pair_spec,
            scratch_shapes=[
                pltpu.VMEM((3 * C, span), jnp.bfloat16),
                pltpu.VMEM((3 * C, span), jnp.bfloat16),
                pltpu.VMEM((C, H * 2 * W), jnp.float32),
            ],
        ),
        compiler_params=pltpu.CompilerParams(
            dimension_semantics=("parallel",)),
    )(x_pairs, w1_s, b1_c, w2_s, b2_c)

    return out.reshape(N, C, H, W)


# speedup vs baseline: 1.1379x; 1.0077x over previous
import functools

import jax
import jax.numpy as jnp
from jax import lax
from jax.experimental import pallas as pl
from jax.experimental.pallas import tpu as pltpu

KS = 3
PAD = 128
NCHUNK = 4


def _resblock_kernel(x_ref, w1_ref, b1_ref, w2_ref, b2_ref,
                     o_ref, s1_ref, s2_ref, *, H, W):
    C = x_ref.shape[2]
    W2 = 2 * W
    L = H * W2
    CH = L // NCHUNK
    HH = CH // W2
    bf16 = jnp.bfloat16

    lmod = lax.broadcasted_iota(jnp.int32, (1, CH), 1) % W
    ok_left = lmod != 0
    ok_right = lmod != (W - 1)

    zb = jnp.zeros((3 * C, PAD), bf16)
    for s in (s1_ref, s2_ref):
        s[:, :PAD] = zb
        s[:, PAD + L:] = zb

    def build_shifted(s_ref):
        for j in range(NCHUNK):
            lo = PAD + j * CH
            left = jnp.where(ok_left, s_ref[C:2 * C, lo - 1:lo - 1 + CH],
                             bf16(0))
            right = jnp.where(ok_right, s_ref[C:2 * C, lo + 1:lo + 1 + CH],
                              bf16(0))
            s_ref[:C, lo:lo + CH] = left
            s_ref[2 * C:, lo:lo + CH] = right

    def conv_chunk(w_ref, b_ref, s_ref, j):
        lo = PAD + j * CH
        acc = jnp.zeros((C, CH), jnp.float32)
        for dy in range(KS):
            off = (dy - 1) * W2
            acc = acc + jnp.dot(w_ref[dy], s_ref[:, lo + off:lo + off + CH],
                                preferred_element_type=jnp.float32)
        return acc + b_ref[...]

    for p in range(x_ref.shape[0]):
        for j in range(NCHUNK):
            xa = x_ref[p, 0, :, j * HH:(j + 1) * HH, :]
            xb = x_ref[p, 1, :, j * HH:(j + 1) * HH, :]
            xpk = jnp.concatenate([xa, xb], axis=2).reshape(C, CH)
            s1_ref[C:2 * C, PAD + j * CH:PAD + (j + 1) * CH] = xpk.astype(bf16)

        build_shifted(s1_ref)
        for j in range(NCHUNK):
            h = jnp.maximum(conv_chunk(w1_ref, b1_ref, s1_ref, j), 0.0)
            s2_ref[C:2 * C, PAD + j * CH:PAD + (j + 1) * CH] = h.astype(bf16)

        build_shifted(s2_ref)
        for j in range(NCHUNK):
            y = conv_chunk(w2_ref, b2_ref, s2_ref, j)
            lo = PAD + j * CH
            x_res = s1_ref[C:2 * C, lo:lo + CH].astype(jnp.float32)
            y = jnp.maximum(y + x_res, 0.0)
            y3 = y.reshape(C, HH, W2)
            o_ref[p, 0, :, j * HH:(j + 1) * HH, :] = y3[:, :, :W]
            o_ref[p, 1, :, j * HH:(j + 1) * HH, :] = y3[:, :, W:]


def kernel(x_nchw, w1, b1, w2, b2):
    N, C, H, W = x_nchw.shape
    assert w2.shape[-1] == C and N % 2 == 0 and 2 * W % 128 == 0
    assert PAD >= 2 * W and (H * 2 * W) % (NCHUNK * 2 * W) == 0

    NB = N // 2
    P = 2 if NB % 2 == 0 else 1
    x_pairs = x_nchw.reshape(NB, 2, C, H, W)

    def stack_w(w):
        t = jnp.transpose(w, (0, 1, 3, 2))
        t = jnp.transpose(t, (0, 2, 1, 3))
        return t.reshape(KS, C, KS * C).astype(jnp.bfloat16)

    w1_s = stack_w(w1)
    w2_s = stack_w(w2)
    b1_c = b1.reshape(C, 1).astype(jnp.float32)
    b2_c = b2.reshape(C, 1).astype(jnp.float32)

    body = functools.partial(_resblock_kernel, H=H, W=W)
    pair_spec = pl.BlockSpec((P, 2, C, H, W), lambda n: (n, 0, 0, 0, 0))
    span = H * 2 * W + 2 * PAD

    out = pl.pallas_call(
        body,
        out_shape=jax.ShapeDtypeStruct((NB, 2, C, H, W), x_nchw.dtype),
        grid_spec=pltpu.PrefetchScalarGridSpec(
            num_scalar_prefetch=0,
            grid=(NB // P,),
            in_specs=[
                pair_spec,
                pl.BlockSpec((KS, C, KS * C), lambda n: (0, 0, 0)),
                pl.BlockSpec((C, 1), lambda n: (0, 0)),
                pl.BlockSpec((KS, C, KS * C), lambda n: (0, 0, 0)),
                pl.BlockSpec((C, 1), lambda n: (0, 0)),
            ],
            out_specs=pair_spec,
            scratch_shapes=[
                pltpu.VMEM((3 * C, span), jnp.bfloat16),
                pltpu.VMEM((3 * C, span), jnp.bfloat16),
            ],
        ),
        compiler_params=pltpu.CompilerParams(
            dimension_semantics=("parallel",)),
    )(x_pairs, w1_s, b1_c, w2_s, b2_c)

    return out.reshape(N, C, H, W)
